# baseline (device time: 298953 ns/iter reference)
import jax
import jax.numpy as jnp
from jax import lax
from jax.experimental import pallas as pl
from jax.experimental.pallas import tpu as pltpu

N_Y = 4


def kernel(O, Wo):
    B, S, Hl, D = O.shape
    K = Hl * D
    N = Wo.shape[1]
    S_out = S // N_Y
    M = B * S_out

    O2 = O.reshape(B, S, K)

    def body(o_ref, w_ref, out_ref, comm_ref, acc_ref, send_sems, recv_sems):
        my_x = lax.axis_index("x")
        my_y = lax.axis_index("y")
        my_z = lax.axis_index("z")
        right = lax.rem(my_y + 1, N_Y)
        left = lax.rem(my_y + N_Y - 1, N_Y)

        barrier_sem = pltpu.get_barrier_semaphore()
        for nbr in (left, right):
            pl.semaphore_signal(
                barrier_sem, inc=1,
                device_id=(my_x, nbr, my_z),
                device_id_type=pl.DeviceIdType.MESH,
            )
        pl.semaphore_wait(barrier_sem, 2)

        def chunk_partial(c):
            o = o_ref[:, pl.ds(c * S_out, S_out), :]
            o2 = o.reshape(M, K)
            return jnp.dot(o2, w_ref[...], preferred_element_type=jnp.float32)

        acc_ref[...] = chunk_partial(lax.rem(my_y + N_Y - 1, N_Y))

        for s in range(N_Y - 1):
            rdma = pltpu.make_async_remote_copy(
                src_ref=acc_ref,
                dst_ref=comm_ref.at[s],
                send_sem=send_sems.at[s],
                recv_sem=recv_sems.at[s],
                device_id=(my_x, right, my_z),
                device_id_type=pl.DeviceIdType.MESH,
            )
            rdma.start()
            rdma.wait()
            c = lax.rem(my_y + 2 * N_Y - 2 - s, N_Y)
            acc_ref[...] = chunk_partial(c) + comm_ref[s]

        out_ref[...] = acc_ref[...].reshape(B, S_out, N)

    return pl.pallas_call(
        body,
        out_shape=jax.ShapeDtypeStruct((B, S_out, N), jnp.float32),
        in_specs=[
            pl.BlockSpec(memory_space=pltpu.VMEM),
            pl.BlockSpec(memory_space=pltpu.VMEM),
        ],
        out_specs=pl.BlockSpec(memory_space=pltpu.VMEM),
        scratch_shapes=[
            pltpu.VMEM((N_Y - 1, M, N), jnp.float32),
            pltpu.VMEM((M, N), jnp.float32),
            pltpu.SemaphoreType.DMA((N_Y - 1,)),
            pltpu.SemaphoreType.DMA((N_Y - 1,)),
        ],
        compiler_params=pltpu.CompilerParams(collective_id=0),
    )(O2, Wo)
